# Initial kernel scaffold; baseline (speedup 1.0000x reference)
import jax
import jax.numpy as jnp
from jax.experimental import pallas as pl


def kernel(x, z, edge_attr, W, b, W1, b1, edge_index):
    raise NotImplementedError("write your pallas kernel here")


# speedup vs baseline: 4.5156x; 4.5156x over previous
import functools

import jax
import jax.numpy as jnp
from jax import lax
from jax.experimental import pallas as pl
from jax.experimental.pallas import tpu as pltpu
from jax.experimental.pallas import tpu_sc as plsc

N = 50000
E = 800000
D_IN = 512
D = 64

SPLIT = 25088
ROWS_T = SPLIT // 16
ACC_ROWS = SPLIT + 8
E_TILE = E // 16
K = 400
N_CHUNK = E_TILE // K


def _mm_body(x_ref, w_ref, b_ref, g_ref, xl_ref, ec_ref):
    xv = x_ref[...]
    xl_ref[...] = (
        jnp.dot(xv, w_ref[...], preferred_element_type=jnp.float32)
        + b_ref[...]
    )
    ec_ref[...] = jnp.dot(xv, g_ref[...], preferred_element_type=jnp.float32)


def _div_body(s_ref, c_ref, o_ref):
    cnt = jnp.maximum(c_ref[...], 1.0)
    o_ref[...] = s_ref[...] / cnt


def _sc_body(xl_hbm, src_hbm, dst_hbm, zer_hbm, zc_hbm,
             sum_hbm, cnt_hbm,
             acc_sh, cnt_sh, srcv, dstv, dstl2, rows_v, ones_v, sem):
    c = lax.axis_index("c")
    s = lax.axis_index("s")
    base = c * SPLIT

    for k in range(K // 16):
        ones_v[pl.ds(k * 16, 16)] = jnp.ones((16,), jnp.float32)

    pltpu.sync_copy(zer_hbm.at[pl.ds(s * ROWS_T, ROWS_T)],
                    acc_sh.at[pl.ds(s * ROWS_T, ROWS_T)])
    pltpu.sync_copy(zc_hbm.at[pl.ds(s * ROWS_T, ROWS_T)],
                    cnt_sh.at[pl.ds(s * ROWS_T, ROWS_T)])
    plsc.subcore_barrier()

    def chunk(i, carry):
        e0 = s * E_TILE + i * K
        pltpu.sync_copy(src_hbm.at[pl.ds(e0, K)], srcv)
        pltpu.sync_copy(dst_hbm.at[pl.ds(e0, K)], dstv)
        cps = []
        for off, sz in ((0, 128), (128, 128), (256, 128), (384, 16)):
            cps.append(pltpu.async_copy(
                xl_hbm.at[srcv.at[pl.ds(off, sz)]],
                rows_v.at[pl.ds(off, sz)], sem))
        for cp in cps:
            cp.wait()
        for k in range(K // 16):
            v = dstv[pl.ds(k * 16, 16)]
            dl = v - base
            keep = (dl >= 0) & (dl < SPLIT)
            dl = jnp.where(keep, dl, SPLIT)
            dstl2[k // 5, pl.ds((k % 5) * 16, 16)] = dl
        for j in range(K // 80):
            pltpu.sync_copy(rows_v.at[pl.ds(j * 80, 80)],
                            acc_sh.at[dstl2.at[j]], add=True)
            pltpu.sync_copy(ones_v.at[pl.ds(j * 80, 80)],
                            cnt_sh.at[dstl2.at[j]], add=True)
        return carry

    lax.fori_loop(0, N_CHUNK, chunk, 0)
    plsc.subcore_barrier()

    pltpu.sync_copy(acc_sh.at[pl.ds(s * ROWS_T, ROWS_T)],
                    sum_hbm.at[pl.ds(base + s * ROWS_T, ROWS_T)])
    pltpu.sync_copy(cnt_sh.at[pl.ds(s * ROWS_T, ROWS_T)],
                    cnt_hbm.at[pl.ds(base + s * ROWS_T, ROWS_T)])


_sc_call = functools.partial(
    pl.kernel,
    out_type=(
        jax.ShapeDtypeStruct((2 * SPLIT, D), jnp.float32),
        jax.ShapeDtypeStruct((2 * SPLIT,), jnp.float32),
    ),
    mesh=plsc.VectorSubcoreMesh(core_axis_name="c", subcore_axis_name="s"),
    scratch_types=[
        pltpu.VMEM_SHARED((ACC_ROWS, D), jnp.float32),
        pltpu.VMEM_SHARED((ACC_ROWS,), jnp.float32),
        pltpu.VMEM((K,), jnp.int32),
        pltpu.VMEM((K,), jnp.int32),
        pltpu.VMEM((K // 80, 80), jnp.int32),
        pltpu.VMEM((K, D), jnp.float32),
        pltpu.VMEM((K,), jnp.float32),
        pltpu.SemaphoreType.DMA,
    ],
    compiler_params=pltpu.CompilerParams(use_tc_tiling_on_sc=False),
)(_sc_body)


def kernel(x, z, edge_attr, W, b, W1, b1, edge_index):
    eg1 = z[0, 0] * jax.nn.relu(edge_attr[0, 0] * W1[0] + b1)

    xl, ec = pl.pallas_call(
        _mm_body,
        grid=(50,),
        in_specs=[
            pl.BlockSpec((N // 50, D_IN), lambda i: (i, 0)),
            pl.BlockSpec((D_IN, D), lambda i: (0, 0)),
            pl.BlockSpec((1, D), lambda i: (0, 0)),
            pl.BlockSpec((D_IN, 1), lambda i: (0, 0)),
        ],
        out_specs=[
            pl.BlockSpec((N // 50, D), lambda i: (i, 0)),
            pl.BlockSpec((N // 50, 1), lambda i: (i, 0)),
        ],
        out_shape=[
            jax.ShapeDtypeStruct((N, D), jnp.float32),
            jax.ShapeDtypeStruct((N, 1), jnp.float32),
        ],
    )(x, W, b.reshape(1, D), eg1.reshape(D_IN, 1))

    src = edge_index[0].astype(jnp.int32)
    dst = edge_index[1].astype(jnp.int32)
    zer = jnp.zeros((SPLIT, D), jnp.float32)
    zc = jnp.zeros((SPLIT,), jnp.float32)

    sums, cnt = _sc_call(xl, src, dst, zer, zc)

    out = pl.pallas_call(
        _div_body,
        grid=(125,),
        in_specs=[
            pl.BlockSpec((400, D), lambda i: (i, 0)),
            pl.BlockSpec((400, 1), lambda i: (i, 0)),
        ],
        out_specs=pl.BlockSpec((400, D), lambda i: (i, 0)),
        out_shape=jax.ShapeDtypeStruct((N, D), jnp.float32),
    )(sums, cnt.reshape(2 * SPLIT, 1))

    eg2 = ec.reshape(1, N)
    return (out, eg2)
